# Initial kernel scaffold; baseline (speedup 1.0000x reference)
import jax
import jax.numpy as jnp
from jax.experimental import pallas as pl


def kernel(usr_x, itm_x, usr_edge_index, itm_edge_index):
    raise NotImplementedError("write your pallas kernel here")


# speedup vs baseline: 8.8644x; 8.8644x over previous
import functools

import jax
import jax.numpy as jnp
from jax import lax
from jax.experimental import pallas as pl
from jax.experimental.pallas import tpu as pltpu
from jax.experimental.pallas import tpu_sc as plsc

N = 10000
NP = 10240
D = 128
DH = 64
E = 320000
NC, NS = 2, 16
CPB = 8
ROWS_PER_TILE = 160
BLKS = ROWS_PER_TILE // CPB
EPAD = ROWS_PER_TILE * 128 * NS
ROWS2D = EPAD // 128
RPT = 640


def _mesh():
    return plsc.VectorSubcoreMesh(
        core_axis_name="c", subcore_axis_name="s", num_cores=NC, num_subcores=NS
    )


def _sc_degrees(us2d, ud2d, is2d, id2d):
    @functools.partial(
        pl.kernel,
        out_type=tuple(jax.ShapeDtypeStruct((NP,), jnp.float32) for _ in range(4)),
        mesh=_mesh(),
        scratch_types=[
            pltpu.VMEM((CPB, 128), jnp.int32),
            pltpu.VMEM((CPB, 128), jnp.int32),
            pltpu.VMEM((128,), jnp.float32),
            pltpu.VMEM((RPT,), jnp.float32),
            pltpu.VMEM_SHARED((NP,), jnp.float32),
            pltpu.VMEM_SHARED((NP,), jnp.float32),
        ],
    )
    def k(us_r, ud_r, is_r, id_r, dus_r, dud_r, dis_r, did_r,
          six, dix, ones_v, stage_v, hist_s, hist_d):
        c = lax.axis_index("c")
        s = lax.axis_index("s")
        for kk in range(8):
            ones_v[pl.ds(kk * 16, 16)] = jnp.ones((16,), jnp.float32)
        for kk in range(RPT // 16):
            stage_v[pl.ds(kk * 16, 16)] = jnp.zeros((16,), jnp.float32)
        off = s * RPT
        pltpu.sync_copy(stage_v, hist_s.at[pl.ds(off, RPT)])
        pltpu.sync_copy(stage_v, hist_d.at[pl.ds(off, RPT)])
        plsc.subcore_barrier()

        def run(src2d, dst2d):
            base = s * ROWS_PER_TILE

            def blk(b, carry):
                r0 = base + b * CPB
                pltpu.sync_copy(src2d.at[pl.ds(r0, CPB)], six)
                pltpu.sync_copy(dst2d.at[pl.ds(r0, CPB)], dix)
                for j in range(CPB):
                    pltpu.sync_copy(ones_v, hist_s.at[six.at[j]], add=True)
                    pltpu.sync_copy(ones_v, hist_d.at[dix.at[j]], add=True)
                return carry

            lax.fori_loop(0, BLKS, blk, 0)

        @pl.when(c == 0)
        def _():
            run(us_r, ud_r)

        @pl.when(c == 1)
        def _():
            run(is_r, id_r)

        plsc.subcore_barrier()

        def wout(hist, out_r):
            pltpu.sync_copy(hist.at[pl.ds(off, RPT)], stage_v)
            pltpu.sync_copy(stage_v, out_r.at[pl.ds(off, RPT)])

        @pl.when(c == 0)
        def _():
            wout(hist_s, dus_r)
            wout(hist_d, dud_r)

        @pl.when(c == 1)
        def _():
            wout(hist_s, dis_r)
            wout(hist_d, did_r)

    return k(us2d, ud2d, is2d, id2d)


def _sc_prop(tu0, tu1, ti0, ti1, us2d, ud2d, is2d, id2d, z320):
    half = jax.ShapeDtypeStruct((NP, DH), jnp.float32)

    @functools.partial(
        pl.kernel,
        out_type=(half, half, half, half),
        mesh=_mesh(),
        compiler_params=pltpu.CompilerParams(use_tc_tiling_on_sc=False),
        scratch_types=[
            pltpu.VMEM((CPB, 128), jnp.int32),
            pltpu.VMEM((CPB, 128), jnp.int32),
            pltpu.VMEM((128, DH), jnp.float32),
            pltpu.VMEM((320, DH), jnp.float32),
            pltpu.VMEM_SHARED((NP, DH), jnp.float32),
        ],
    )
    def k(tu0_r, tu1_r, ti0_r, ti1_r, us_r, ud_r, is_r, id_r, z_r,
          accu0_r, accu1_r, acci0_r, acci1_r, six, dix, msg, stage, acc_s):
        c = lax.axis_index("c")
        s = lax.axis_index("s")
        off = s * RPT
        base = s * ROWS_PER_TILE

        def zero_acc():
            pltpu.sync_copy(z_r, stage)
            pltpu.sync_copy(stage, acc_s.at[pl.ds(off, 320)])
            pltpu.sync_copy(stage, acc_s.at[pl.ds(off + 320, 320)])

        def scatter_pass(tbl, src2d, dst2d):
            def blk(b, carry):
                r0 = base + b * CPB
                pltpu.sync_copy(src2d.at[pl.ds(r0, CPB)], six)
                pltpu.sync_copy(dst2d.at[pl.ds(r0, CPB)], dix)
                for j in range(CPB):
                    pltpu.sync_copy(tbl.at[six.at[j]], msg)
                    pltpu.sync_copy(msg, acc_s.at[dix.at[j]], add=True)
                return carry

            lax.fori_loop(0, BLKS, blk, 0)

        def wout(out_r):
            for h in range(2):
                o = off + h * 320
                pltpu.sync_copy(acc_s.at[pl.ds(o, 320)], stage)
                pltpu.sync_copy(stage, out_r.at[pl.ds(o, 320)])

        def direction(tbl, src2d, dst2d, out_r):
            zero_acc()
            plsc.subcore_barrier()
            scatter_pass(tbl, src2d, dst2d)
            plsc.subcore_barrier()
            wout(out_r)

        @pl.when(c == 0)
        def _():
            direction(tu0_r, us_r, ud_r, acci0_r)
            direction(ti0_r, is_r, id_r, accu0_r)

        @pl.when(c == 1)
        def _():
            direction(tu1_r, us_r, ud_r, acci1_r)
            direction(ti1_r, is_r, id_r, accu1_r)

    return k(tu0, tu1, ti0, ti1, us2d, ud2d, is2d, id2d, z320)


_GRID = NP // RPT
_MATH = pl.BlockSpec((RPT, DH), lambda i: (i, 0))
_VEC = pl.BlockSpec((RPT, 1), lambda i: (i, 0))
_HALF = jax.ShapeDtypeStruct((NP, DH), jnp.float32)
_VECS = jax.ShapeDtypeStruct((NP, 1), jnp.float32)


def _tc_scales(dus, dud, dis, did, ux0, ux1, ix0, ix1):
    def body(dus_r, dud_r, dis_r, did_r, ux0_r, ux1_r, ix0_r, ix1_r,
             tu0_o, tu1_o, ti0_o, ti1_o, bu_o, bi_o, su_o, si_o,
             pu0_o, pu1_o, pi0_o, pi1_o):
        rs = lambda v: lax.rsqrt(jnp.maximum(v, 1.0))
        a_u = rs(dus_r[...])
        b_i = rs(dud_r[...])
        a_i = rs(dis_r[...])
        b_u = rs(did_r[...])
        tu0_o[...] = a_u * ux0_r[...]
        tu1_o[...] = a_u * ux1_r[...]
        ti0_o[...] = a_i * ix0_r[...]
        ti1_o[...] = a_i * ix1_r[...]
        bu_o[...] = b_u
        bi_o[...] = b_i
        su_o[...] = a_u * b_u
        si_o[...] = a_i * b_i
        pu0_o[...] = 0.25 * ux0_r[...]
        pu1_o[...] = 0.25 * ux1_r[...]
        pi0_o[...] = 0.25 * ix0_r[...]
        pi1_o[...] = 0.25 * ix1_r[...]

    return pl.pallas_call(
        body,
        grid=(_GRID,),
        in_specs=[_VEC, _VEC, _VEC, _VEC, _MATH, _MATH, _MATH, _MATH],
        out_specs=(_MATH,) * 4 + (_VEC,) * 4 + (_MATH,) * 4,
        out_shape=(_HALF,) * 4 + (_VECS,) * 4 + (_HALF,) * 4,
    )(dus, dud, dis, did, ux0, ux1, ix0, ix1)


def _tc_rescale(accu0, accu1, acci0, acci1, bu, bi, su, si,
                pu0, pu1, pi0, pi1, with_tables):
    def body(au0_r, au1_r, ai0_r, ai1_r, bu_r, bi_r, su_r, si_r,
             pu0_r, pu1_r, pi0_r, pi1_r, *outs):
        bu_v, bi_v = bu_r[...], bi_r[...]
        outs[0][...] = pu0_r[...] + 0.25 * (bu_v * au0_r[...])
        outs[1][...] = pu1_r[...] + 0.25 * (bu_v * au1_r[...])
        outs[2][...] = pi0_r[...] + 0.25 * (bi_v * ai0_r[...])
        outs[3][...] = pi1_r[...] + 0.25 * (bi_v * ai1_r[...])
        if with_tables:
            su_v, si_v = su_r[...], si_r[...]
            outs[4][...] = su_v * au0_r[...]
            outs[5][...] = su_v * au1_r[...]
            outs[6][...] = si_v * ai0_r[...]
            outs[7][...] = si_v * ai1_r[...]

    n_out = 8 if with_tables else 4
    return pl.pallas_call(
        body,
        grid=(_GRID,),
        in_specs=[_MATH] * 4 + [_VEC] * 4 + [_MATH] * 4,
        out_specs=(_MATH,) * n_out,
        out_shape=(_HALF,) * n_out,
    )(accu0, accu1, acci0, acci1, bu, bi, su, si, pu0, pu1, pi0, pi1)


def kernel(usr_x, itm_x, usr_edge_index, itm_edge_index):
    uxp = jnp.pad(usr_x, ((0, NP - N), (0, 0)))
    ixp = jnp.pad(itm_x, ((0, NP - N), (0, 0)))
    ux0, ux1 = uxp[:, :DH], uxp[:, DH:]
    ix0, ix1 = ixp[:, :DH], ixp[:, DH:]
    pad = N + (jnp.arange(EPAD - E, dtype=jnp.int32) % (NP - N))

    def prep(row):
        return jnp.concatenate([row, pad]).reshape(ROWS2D, 128)

    us2d, ud2d = prep(usr_edge_index[0]), prep(usr_edge_index[1])
    is2d, id2d = prep(itm_edge_index[0]), prep(itm_edge_index[1])
    z320 = jnp.zeros((320, DH), jnp.float32)

    dus, dud, dis, did = _sc_degrees(us2d, ud2d, is2d, id2d)
    (tu0, tu1, ti0, ti1, bu, bi, su, si,
     pu0, pu1, pi0, pi1) = _tc_scales(
        dus.reshape(NP, 1), dud.reshape(NP, 1),
        dis.reshape(NP, 1), did.reshape(NP, 1), ux0, ux1, ix0, ix1)

    for r in range(3):
        accu0, accu1, acci0, acci1 = _sc_prop(
            tu0, tu1, ti0, ti1, us2d, ud2d, is2d, id2d, z320)
        if r < 2:
            (pu0, pu1, pi0, pi1, tu0, tu1, ti0, ti1) = _tc_rescale(
                accu0, accu1, acci0, acci1, bu, bi, su, si,
                pu0, pu1, pi0, pi1, True)
        else:
            pu0, pu1, pi0, pi1 = _tc_rescale(
                accu0, accu1, acci0, acci1, bu, bi, su, si,
                pu0, pu1, pi0, pi1, False)

    new_usr = jnp.concatenate([pu0, pu1], axis=1)[:N]
    new_itm = jnp.concatenate([pi0, pi1], axis=1)[:N]
    return new_usr, new_itm


# speedup vs baseline: 12.8099x; 1.4451x over previous
import functools

import jax
import jax.numpy as jnp
from jax import lax
from jax.experimental import pallas as pl
from jax.experimental.pallas import tpu as pltpu
from jax.experimental.pallas import tpu_sc as plsc

N = 10000
NP = 10240
D = 128
DH = 64
E = 320000
NC, NS = 2, 16
CPB = 8
ROWS_PER_TILE = 160
BLKS = ROWS_PER_TILE // CPB
EPAD = ROWS_PER_TILE * 128 * NS
ROWS2D = EPAD // 128
RPT = 640


def _mesh():
    return plsc.VectorSubcoreMesh(
        core_axis_name="c", subcore_axis_name="s", num_cores=NC, num_subcores=NS
    )


def _sc_degrees(us2d, ud2d, is2d, id2d):
    @functools.partial(
        pl.kernel,
        out_type=tuple(jax.ShapeDtypeStruct((NP,), jnp.float32) for _ in range(4)),
        mesh=_mesh(),
        scratch_types=[
            pltpu.VMEM((CPB, 128), jnp.int32),
            pltpu.VMEM((CPB, 128), jnp.int32),
            pltpu.VMEM((128,), jnp.float32),
            pltpu.VMEM((RPT,), jnp.float32),
            pltpu.VMEM_SHARED((NP,), jnp.float32),
            pltpu.VMEM_SHARED((NP,), jnp.float32),
        ],
    )
    def k(us_r, ud_r, is_r, id_r, dus_r, dud_r, dis_r, did_r,
          six, dix, ones_v, stage_v, hist_s, hist_d):
        c = lax.axis_index("c")
        s = lax.axis_index("s")
        for kk in range(8):
            ones_v[pl.ds(kk * 16, 16)] = jnp.ones((16,), jnp.float32)
        for kk in range(RPT // 16):
            stage_v[pl.ds(kk * 16, 16)] = jnp.zeros((16,), jnp.float32)
        off = s * RPT
        pltpu.sync_copy(stage_v, hist_s.at[pl.ds(off, RPT)])
        pltpu.sync_copy(stage_v, hist_d.at[pl.ds(off, RPT)])
        plsc.subcore_barrier()

        def run(src2d, dst2d):
            base = s * ROWS_PER_TILE

            def blk(b, carry):
                r0 = base + b * CPB
                pltpu.sync_copy(src2d.at[pl.ds(r0, CPB)], six)
                pltpu.sync_copy(dst2d.at[pl.ds(r0, CPB)], dix)
                for j in range(CPB):
                    pltpu.sync_copy(ones_v, hist_s.at[six.at[j]], add=True)
                    pltpu.sync_copy(ones_v, hist_d.at[dix.at[j]], add=True)
                return carry

            lax.fori_loop(0, BLKS, blk, 0)

        @pl.when(c == 0)
        def _():
            run(us_r, ud_r)

        @pl.when(c == 1)
        def _():
            run(is_r, id_r)

        plsc.subcore_barrier()

        def wout(hist, out_r):
            pltpu.sync_copy(hist.at[pl.ds(off, RPT)], stage_v)
            pltpu.sync_copy(stage_v, out_r.at[pl.ds(off, RPT)])

        @pl.when(c == 0)
        def _():
            wout(hist_s, dus_r)
            wout(hist_d, dud_r)

        @pl.when(c == 1)
        def _():
            wout(hist_s, dis_r)
            wout(hist_d, did_r)

    return k(us2d, ud2d, is2d, id2d)


def _sc_prop(tu0, tu1, ti0, ti1, us2d, ud2d, is2d, id2d, z320):
    half = jax.ShapeDtypeStruct((NP, DH), jnp.float32)

    @functools.partial(
        pl.kernel,
        out_type=(half, half, half, half),
        mesh=_mesh(),
        compiler_params=pltpu.CompilerParams(use_tc_tiling_on_sc=False),
        scratch_types=[
            pltpu.VMEM((CPB, 128), jnp.int32),
            pltpu.VMEM((CPB, 128), jnp.int32),
            pltpu.VMEM((4 * 128, DH), jnp.float32),
            pltpu.VMEM((320, DH), jnp.float32),
            pltpu.VMEM_SHARED((NP, DH), jnp.float32),
            pltpu.SemaphoreType.DMA,
            pltpu.SemaphoreType.DMA,
            pltpu.SemaphoreType.DMA,
            pltpu.SemaphoreType.DMA,
            pltpu.SemaphoreType.DMA,
            pltpu.SemaphoreType.DMA,
            pltpu.SemaphoreType.DMA,
            pltpu.SemaphoreType.DMA,
        ],
    )
    def k(tu0_r, tu1_r, ti0_r, ti1_r, us_r, ud_r, is_r, id_r, z_r,
          accu0_r, accu1_r, acci0_r, acci1_r, six, dix, msg, stage, acc_s,
          g0, g1, g2, g3, s0, s1, s2, s3):
        c = lax.axis_index("c")
        s = lax.axis_index("s")
        off = s * RPT
        base = s * ROWS_PER_TILE
        gsem = (g0, g1, g2, g3)
        ssem = (s0, s1, s2, s3)

        def zero_acc():
            pltpu.sync_copy(z_r, stage)
            pltpu.sync_copy(stage, acc_s.at[pl.ds(off, 320)])
            pltpu.sync_copy(stage, acc_s.at[pl.ds(off + 320, 320)])

        def scatter_pass(tbl, src2d, dst2d):
            def mslice(j):
                return msg.at[pl.ds((j % 4) * 128, 128)]

            def blk(b, carry):
                r0 = base + b * CPB
                pltpu.sync_copy(src2d.at[pl.ds(r0, CPB)], six)
                pltpu.sync_copy(dst2d.at[pl.ds(r0, CPB)], dix)
                gd = [None] * CPB
                sd = [None] * CPB
                for j in range(2):
                    gd[j] = pltpu.async_copy(tbl.at[six.at[j]], mslice(j), gsem[j % 4])
                for j in range(CPB):
                    gd[j].wait()
                    sd[j] = pltpu.async_copy(
                        mslice(j), acc_s.at[dix.at[j]], ssem[j % 4], add=True)
                    nj = j + 2
                    if nj < CPB:
                        if nj >= 4:
                            sd[nj - 4].wait()
                        gd[nj] = pltpu.async_copy(
                            tbl.at[six.at[nj]], mslice(nj), gsem[nj % 4])
                for j in range(CPB - 4, CPB):
                    sd[j].wait()
                return carry

            lax.fori_loop(0, BLKS, blk, 0)

        def wout(out_r):
            for h in range(2):
                o = off + h * 320
                pltpu.sync_copy(acc_s.at[pl.ds(o, 320)], stage)
                pltpu.sync_copy(stage, out_r.at[pl.ds(o, 320)])

        def direction(tbl, src2d, dst2d, out_r):
            zero_acc()
            plsc.subcore_barrier()
            scatter_pass(tbl, src2d, dst2d)
            plsc.subcore_barrier()
            wout(out_r)

        @pl.when(c == 0)
        def _():
            direction(tu0_r, us_r, ud_r, acci0_r)
            direction(ti0_r, is_r, id_r, accu0_r)

        @pl.when(c == 1)
        def _():
            direction(tu1_r, us_r, ud_r, acci1_r)
            direction(ti1_r, is_r, id_r, accu1_r)

    return k(tu0, tu1, ti0, ti1, us2d, ud2d, is2d, id2d, z320)


_GRID = NP // RPT
_MATH = pl.BlockSpec((RPT, DH), lambda i: (i, 0))
_VEC = pl.BlockSpec((RPT, 1), lambda i: (i, 0))
_HALF = jax.ShapeDtypeStruct((NP, DH), jnp.float32)
_VECS = jax.ShapeDtypeStruct((NP, 1), jnp.float32)


def _tc_scales(dus, dud, dis, did, ux0, ux1, ix0, ix1):
    def body(dus_r, dud_r, dis_r, did_r, ux0_r, ux1_r, ix0_r, ix1_r,
             tu0_o, tu1_o, ti0_o, ti1_o, bu_o, bi_o, su_o, si_o,
             pu0_o, pu1_o, pi0_o, pi1_o):
        rs = lambda v: lax.rsqrt(jnp.maximum(v, 1.0))
        a_u = rs(dus_r[...])
        b_i = rs(dud_r[...])
        a_i = rs(dis_r[...])
        b_u = rs(did_r[...])
        tu0_o[...] = a_u * ux0_r[...]
        tu1_o[...] = a_u * ux1_r[...]
        ti0_o[...] = a_i * ix0_r[...]
        ti1_o[...] = a_i * ix1_r[...]
        bu_o[...] = b_u
        bi_o[...] = b_i
        su_o[...] = a_u * b_u
        si_o[...] = a_i * b_i
        pu0_o[...] = 0.25 * ux0_r[...]
        pu1_o[...] = 0.25 * ux1_r[...]
        pi0_o[...] = 0.25 * ix0_r[...]
        pi1_o[...] = 0.25 * ix1_r[...]

    return pl.pallas_call(
        body,
        grid=(_GRID,),
        in_specs=[_VEC, _VEC, _VEC, _VEC, _MATH, _MATH, _MATH, _MATH],
        out_specs=(_MATH,) * 4 + (_VEC,) * 4 + (_MATH,) * 4,
        out_shape=(_HALF,) * 4 + (_VECS,) * 4 + (_HALF,) * 4,
    )(dus, dud, dis, did, ux0, ux1, ix0, ix1)


def _tc_rescale(accu0, accu1, acci0, acci1, bu, bi, su, si,
                pu0, pu1, pi0, pi1, with_tables):
    def body(au0_r, au1_r, ai0_r, ai1_r, bu_r, bi_r, su_r, si_r,
             pu0_r, pu1_r, pi0_r, pi1_r, *outs):
        bu_v, bi_v = bu_r[...], bi_r[...]
        outs[0][...] = pu0_r[...] + 0.25 * (bu_v * au0_r[...])
        outs[1][...] = pu1_r[...] + 0.25 * (bu_v * au1_r[...])
        outs[2][...] = pi0_r[...] + 0.25 * (bi_v * ai0_r[...])
        outs[3][...] = pi1_r[...] + 0.25 * (bi_v * ai1_r[...])
        if with_tables:
            su_v, si_v = su_r[...], si_r[...]
            outs[4][...] = su_v * au0_r[...]
            outs[5][...] = su_v * au1_r[...]
            outs[6][...] = si_v * ai0_r[...]
            outs[7][...] = si_v * ai1_r[...]

    n_out = 8 if with_tables else 4
    return pl.pallas_call(
        body,
        grid=(_GRID,),
        in_specs=[_MATH] * 4 + [_VEC] * 4 + [_MATH] * 4,
        out_specs=(_MATH,) * n_out,
        out_shape=(_HALF,) * n_out,
    )(accu0, accu1, acci0, acci1, bu, bi, su, si, pu0, pu1, pi0, pi1)


def kernel(usr_x, itm_x, usr_edge_index, itm_edge_index):
    uxp = jnp.pad(usr_x, ((0, NP - N), (0, 0)))
    ixp = jnp.pad(itm_x, ((0, NP - N), (0, 0)))
    ux0, ux1 = uxp[:, :DH], uxp[:, DH:]
    ix0, ix1 = ixp[:, :DH], ixp[:, DH:]
    pad = N + (jnp.arange(EPAD - E, dtype=jnp.int32) % (NP - N))

    def prep(row):
        return jnp.concatenate([row, pad]).reshape(ROWS2D, 128)

    us2d, ud2d = prep(usr_edge_index[0]), prep(usr_edge_index[1])
    is2d, id2d = prep(itm_edge_index[0]), prep(itm_edge_index[1])
    z320 = jnp.zeros((320, DH), jnp.float32)

    dus, dud, dis, did = _sc_degrees(us2d, ud2d, is2d, id2d)
    (tu0, tu1, ti0, ti1, bu, bi, su, si,
     pu0, pu1, pi0, pi1) = _tc_scales(
        dus.reshape(NP, 1), dud.reshape(NP, 1),
        dis.reshape(NP, 1), did.reshape(NP, 1), ux0, ux1, ix0, ix1)

    for r in range(3):
        accu0, accu1, acci0, acci1 = _sc_prop(
            tu0, tu1, ti0, ti1, us2d, ud2d, is2d, id2d, z320)
        if r < 2:
            (pu0, pu1, pi0, pi1, tu0, tu1, ti0, ti1) = _tc_rescale(
                accu0, accu1, acci0, acci1, bu, bi, su, si,
                pu0, pu1, pi0, pi1, True)
        else:
            pu0, pu1, pi0, pi1 = _tc_rescale(
                accu0, accu1, acci0, acci1, bu, bi, su, si,
                pu0, pu1, pi0, pi1, False)

    new_usr = jnp.concatenate([pu0, pu1], axis=1)[:N]
    new_itm = jnp.concatenate([pi0, pi1], axis=1)[:N]
    return new_usr, new_itm


# speedup vs baseline: 12.9557x; 1.0114x over previous
import functools

import jax
import jax.numpy as jnp
from jax import lax
from jax.experimental import pallas as pl
from jax.experimental.pallas import tpu as pltpu
from jax.experimental.pallas import tpu_sc as plsc

N = 10000
NP = 10240
D = 128
DH = 64
E = 320000
NC, NS = 2, 16
CPB = 8
ROWS_PER_TILE = 160
BLKS = ROWS_PER_TILE // CPB
EPAD = ROWS_PER_TILE * 128 * NS
ROWS2D = EPAD // 128
RPT = 640


def _mesh():
    return plsc.VectorSubcoreMesh(
        core_axis_name="c", subcore_axis_name="s", num_cores=NC, num_subcores=NS
    )


def _sc_degrees(us2d, ud2d, is2d, id2d):
    @functools.partial(
        pl.kernel,
        out_type=tuple(jax.ShapeDtypeStruct((NP,), jnp.float32) for _ in range(4)),
        mesh=_mesh(),
        scratch_types=[
            pltpu.VMEM((CPB, 128), jnp.int32),
            pltpu.VMEM((CPB, 128), jnp.int32),
            pltpu.VMEM((128,), jnp.float32),
            pltpu.VMEM((RPT,), jnp.float32),
            pltpu.VMEM_SHARED((NP,), jnp.float32),
            pltpu.VMEM_SHARED((NP,), jnp.float32),
        ],
    )
    def k(us_r, ud_r, is_r, id_r, dus_r, dud_r, dis_r, did_r,
          six, dix, ones_v, stage_v, hist_s, hist_d):
        c = lax.axis_index("c")
        s = lax.axis_index("s")
        for kk in range(8):
            ones_v[pl.ds(kk * 16, 16)] = jnp.ones((16,), jnp.float32)
        for kk in range(RPT // 16):
            stage_v[pl.ds(kk * 16, 16)] = jnp.zeros((16,), jnp.float32)
        off = s * RPT
        pltpu.sync_copy(stage_v, hist_s.at[pl.ds(off, RPT)])
        pltpu.sync_copy(stage_v, hist_d.at[pl.ds(off, RPT)])
        plsc.subcore_barrier()

        def run(src2d, dst2d):
            base = s * ROWS_PER_TILE

            def blk(b, carry):
                r0 = base + b * CPB
                pltpu.sync_copy(src2d.at[pl.ds(r0, CPB)], six)
                pltpu.sync_copy(dst2d.at[pl.ds(r0, CPB)], dix)
                for j in range(CPB):
                    pltpu.sync_copy(ones_v, hist_s.at[six.at[j]], add=True)
                    pltpu.sync_copy(ones_v, hist_d.at[dix.at[j]], add=True)
                return carry

            lax.fori_loop(0, BLKS, blk, 0)

        @pl.when(c == 0)
        def _():
            run(us_r, ud_r)

        @pl.when(c == 1)
        def _():
            run(is_r, id_r)

        plsc.subcore_barrier()

        def wout(hist, out_r):
            pltpu.sync_copy(hist.at[pl.ds(off, RPT)], stage_v)
            pltpu.sync_copy(stage_v, out_r.at[pl.ds(off, RPT)])

        @pl.when(c == 0)
        def _():
            wout(hist_s, dus_r)
            wout(hist_d, dud_r)

        @pl.when(c == 1)
        def _():
            wout(hist_s, dis_r)
            wout(hist_d, did_r)

    return k(us2d, ud2d, is2d, id2d)


def _sc_prop(tu0, tu1, ti0, ti1, us2d, ud2d, is2d, id2d, z320):
    half = jax.ShapeDtypeStruct((NP, DH), jnp.float32)

    @functools.partial(
        pl.kernel,
        out_type=(half, half, half, half),
        mesh=_mesh(),
        compiler_params=pltpu.CompilerParams(use_tc_tiling_on_sc=False),
        scratch_types=[
            pltpu.VMEM((CPB, 128), jnp.int32),
            pltpu.VMEM((CPB, 128), jnp.int32),
            pltpu.VMEM((8 * 128, DH), jnp.float32),
            pltpu.VMEM((320, DH), jnp.float32),
            pltpu.VMEM_SHARED((NP, DH), jnp.float32),
        ] + [pltpu.SemaphoreType.DMA] * 16,
    )
    def k(tu0_r, tu1_r, ti0_r, ti1_r, us_r, ud_r, is_r, id_r, z_r,
          accu0_r, accu1_r, acci0_r, acci1_r, six, dix, msg, stage, acc_s,
          *sems):
        c = lax.axis_index("c")
        s = lax.axis_index("s")
        off = s * RPT
        base = s * ROWS_PER_TILE
        gsem = sems[:8]
        ssem = sems[8:]

        def zero_acc():
            pltpu.sync_copy(z_r, stage)
            pltpu.sync_copy(stage, acc_s.at[pl.ds(off, 320)])
            pltpu.sync_copy(stage, acc_s.at[pl.ds(off + 320, 320)])

        def scatter_pass(tbl, src2d, dst2d):
            def mslice(j):
                return msg.at[pl.ds((j % 8) * 128, 128)]

            def blk(b, carry):
                r0 = base + b * CPB
                pltpu.sync_copy(src2d.at[pl.ds(r0, CPB)], six)
                pltpu.sync_copy(dst2d.at[pl.ds(r0, CPB)], dix)
                gd = [None] * CPB
                sd = [None] * CPB
                for j in range(4):
                    gd[j] = pltpu.async_copy(tbl.at[six.at[j]], mslice(j), gsem[j % 8])
                for j in range(CPB):
                    gd[j].wait()
                    sd[j] = pltpu.async_copy(
                        mslice(j), acc_s.at[dix.at[j]], ssem[j % 8], add=True)
                    nj = j + 4
                    if nj < CPB:
                        gd[nj] = pltpu.async_copy(
                            tbl.at[six.at[nj]], mslice(nj), gsem[nj % 8])
                for j in range(CPB):
                    sd[j].wait()
                return carry

            lax.fori_loop(0, BLKS, blk, 0)

        def wout(out_r):
            for h in range(2):
                o = off + h * 320
                pltpu.sync_copy(acc_s.at[pl.ds(o, 320)], stage)
                pltpu.sync_copy(stage, out_r.at[pl.ds(o, 320)])

        def direction(tbl, src2d, dst2d, out_r):
            zero_acc()
            plsc.subcore_barrier()
            scatter_pass(tbl, src2d, dst2d)
            plsc.subcore_barrier()
            wout(out_r)

        @pl.when(c == 0)
        def _():
            direction(tu0_r, us_r, ud_r, acci0_r)
            direction(ti0_r, is_r, id_r, accu0_r)

        @pl.when(c == 1)
        def _():
            direction(tu1_r, us_r, ud_r, acci1_r)
            direction(ti1_r, is_r, id_r, accu1_r)

    return k(tu0, tu1, ti0, ti1, us2d, ud2d, is2d, id2d, z320)


_GRID = NP // RPT
_MATH = pl.BlockSpec((RPT, DH), lambda i: (i, 0))
_VEC = pl.BlockSpec((RPT, 1), lambda i: (i, 0))
_HALF = jax.ShapeDtypeStruct((NP, DH), jnp.float32)
_VECS = jax.ShapeDtypeStruct((NP, 1), jnp.float32)


def _tc_scales(dus, dud, dis, did, ux0, ux1, ix0, ix1):
    def body(dus_r, dud_r, dis_r, did_r, ux0_r, ux1_r, ix0_r, ix1_r,
             tu0_o, tu1_o, ti0_o, ti1_o, bu_o, bi_o, su_o, si_o,
             pu0_o, pu1_o, pi0_o, pi1_o):
        rs = lambda v: lax.rsqrt(jnp.maximum(v, 1.0))
        a_u = rs(dus_r[...])
        b_i = rs(dud_r[...])
        a_i = rs(dis_r[...])
        b_u = rs(did_r[...])
        tu0_o[...] = a_u * ux0_r[...]
        tu1_o[...] = a_u * ux1_r[...]
        ti0_o[...] = a_i * ix0_r[...]
        ti1_o[...] = a_i * ix1_r[...]
        bu_o[...] = b_u
        bi_o[...] = b_i
        su_o[...] = a_u * b_u
        si_o[...] = a_i * b_i
        pu0_o[...] = 0.25 * ux0_r[...]
        pu1_o[...] = 0.25 * ux1_r[...]
        pi0_o[...] = 0.25 * ix0_r[...]
        pi1_o[...] = 0.25 * ix1_r[...]

    return pl.pallas_call(
        body,
        grid=(_GRID,),
        in_specs=[_VEC, _VEC, _VEC, _VEC, _MATH, _MATH, _MATH, _MATH],
        out_specs=(_MATH,) * 4 + (_VEC,) * 4 + (_MATH,) * 4,
        out_shape=(_HALF,) * 4 + (_VECS,) * 4 + (_HALF,) * 4,
    )(dus, dud, dis, did, ux0, ux1, ix0, ix1)


def _tc_rescale(accu0, accu1, acci0, acci1, bu, bi, su, si,
                pu0, pu1, pi0, pi1, with_tables):
    def body(au0_r, au1_r, ai0_r, ai1_r, bu_r, bi_r, su_r, si_r,
             pu0_r, pu1_r, pi0_r, pi1_r, *outs):
        bu_v, bi_v = bu_r[...], bi_r[...]
        outs[0][...] = pu0_r[...] + 0.25 * (bu_v * au0_r[...])
        outs[1][...] = pu1_r[...] + 0.25 * (bu_v * au1_r[...])
        outs[2][...] = pi0_r[...] + 0.25 * (bi_v * ai0_r[...])
        outs[3][...] = pi1_r[...] + 0.25 * (bi_v * ai1_r[...])
        if with_tables:
            su_v, si_v = su_r[...], si_r[...]
            outs[4][...] = su_v * au0_r[...]
            outs[5][...] = su_v * au1_r[...]
            outs[6][...] = si_v * ai0_r[...]
            outs[7][...] = si_v * ai1_r[...]

    n_out = 8 if with_tables else 4
    return pl.pallas_call(
        body,
        grid=(_GRID,),
        in_specs=[_MATH] * 4 + [_VEC] * 4 + [_MATH] * 4,
        out_specs=(_MATH,) * n_out,
        out_shape=(_HALF,) * n_out,
    )(accu0, accu1, acci0, acci1, bu, bi, su, si, pu0, pu1, pi0, pi1)


def kernel(usr_x, itm_x, usr_edge_index, itm_edge_index):
    uxp = jnp.pad(usr_x, ((0, NP - N), (0, 0)))
    ixp = jnp.pad(itm_x, ((0, NP - N), (0, 0)))
    ux0, ux1 = uxp[:, :DH], uxp[:, DH:]
    ix0, ix1 = ixp[:, :DH], ixp[:, DH:]
    pad = N + (jnp.arange(EPAD - E, dtype=jnp.int32) % (NP - N))

    def prep(row):
        return jnp.concatenate([row, pad]).reshape(ROWS2D, 128)

    us2d, ud2d = prep(usr_edge_index[0]), prep(usr_edge_index[1])
    is2d, id2d = prep(itm_edge_index[0]), prep(itm_edge_index[1])
    z320 = jnp.zeros((320, DH), jnp.float32)

    dus, dud, dis, did = _sc_degrees(us2d, ud2d, is2d, id2d)
    (tu0, tu1, ti0, ti1, bu, bi, su, si,
     pu0, pu1, pi0, pi1) = _tc_scales(
        dus.reshape(NP, 1), dud.reshape(NP, 1),
        dis.reshape(NP, 1), did.reshape(NP, 1), ux0, ux1, ix0, ix1)

    for r in range(3):
        accu0, accu1, acci0, acci1 = _sc_prop(
            tu0, tu1, ti0, ti1, us2d, ud2d, is2d, id2d, z320)
        if r < 2:
            (pu0, pu1, pi0, pi1, tu0, tu1, ti0, ti1) = _tc_rescale(
                accu0, accu1, acci0, acci1, bu, bi, su, si,
                pu0, pu1, pi0, pi1, True)
        else:
            pu0, pu1, pi0, pi1 = _tc_rescale(
                accu0, accu1, acci0, acci1, bu, bi, su, si,
                pu0, pu1, pi0, pi1, False)

    new_usr = jnp.concatenate([pu0, pu1], axis=1)[:N]
    new_itm = jnp.concatenate([pi0, pi1], axis=1)[:N]
    return new_usr, new_itm


# speedup vs baseline: 14.5657x; 1.1243x over previous
import functools

import jax
import jax.numpy as jnp
from jax import lax
from jax.experimental import pallas as pl
from jax.experimental.pallas import tpu as pltpu
from jax.experimental.pallas import tpu_sc as plsc

N = 10000
NP = 10240
D = 128
DH = 64
E = 320000
NC, NS = 2, 16
CPB = 16
ROWS_PER_TILE = 160
BLKS = ROWS_PER_TILE // CPB
EPAD = ROWS_PER_TILE * 128 * NS
ROWS2D = EPAD // 128
RPT = 640


def _mesh():
    return plsc.VectorSubcoreMesh(
        core_axis_name="c", subcore_axis_name="s", num_cores=NC, num_subcores=NS
    )


def _sc_degrees(us2d, ud2d, is2d, id2d):
    @functools.partial(
        pl.kernel,
        out_type=tuple(jax.ShapeDtypeStruct((NP,), jnp.float32) for _ in range(4)),
        mesh=_mesh(),
        scratch_types=[
            pltpu.VMEM((CPB, 128), jnp.int32),
            pltpu.VMEM((CPB, 128), jnp.int32),
            pltpu.VMEM((128,), jnp.float32),
            pltpu.VMEM((RPT,), jnp.float32),
            pltpu.VMEM_SHARED((NP,), jnp.float32),
            pltpu.VMEM_SHARED((NP,), jnp.float32),
        ],
    )
    def k(us_r, ud_r, is_r, id_r, dus_r, dud_r, dis_r, did_r,
          six, dix, ones_v, stage_v, hist_s, hist_d):
        c = lax.axis_index("c")
        s = lax.axis_index("s")
        for kk in range(8):
            ones_v[pl.ds(kk * 16, 16)] = jnp.ones((16,), jnp.float32)
        for kk in range(RPT // 16):
            stage_v[pl.ds(kk * 16, 16)] = jnp.zeros((16,), jnp.float32)
        off = s * RPT
        pltpu.sync_copy(stage_v, hist_s.at[pl.ds(off, RPT)])
        pltpu.sync_copy(stage_v, hist_d.at[pl.ds(off, RPT)])
        plsc.subcore_barrier()

        def run(src2d, dst2d):
            base = s * ROWS_PER_TILE

            def blk(b, carry):
                r0 = base + b * CPB
                pltpu.sync_copy(src2d.at[pl.ds(r0, CPB)], six)
                pltpu.sync_copy(dst2d.at[pl.ds(r0, CPB)], dix)
                for j in range(CPB):
                    pltpu.sync_copy(ones_v, hist_s.at[six.at[j]], add=True)
                    pltpu.sync_copy(ones_v, hist_d.at[dix.at[j]], add=True)
                return carry

            lax.fori_loop(0, BLKS, blk, 0)

        @pl.when(c == 0)
        def _():
            run(us_r, ud_r)

        @pl.when(c == 1)
        def _():
            run(is_r, id_r)

        plsc.subcore_barrier()

        def wout(hist, out_r):
            pltpu.sync_copy(hist.at[pl.ds(off, RPT)], stage_v)
            pltpu.sync_copy(stage_v, out_r.at[pl.ds(off, RPT)])

        @pl.when(c == 0)
        def _():
            wout(hist_s, dus_r)
            wout(hist_d, dud_r)

        @pl.when(c == 1)
        def _():
            wout(hist_s, dis_r)
            wout(hist_d, did_r)

    return k(us2d, ud2d, is2d, id2d)


def _sc_prop(tu0, tu1, ti0, ti1, us2d, ud2d, is2d, id2d, z320):
    half = jax.ShapeDtypeStruct((NP, DH), jnp.float32)

    @functools.partial(
        pl.kernel,
        out_type=(half, half, half, half),
        mesh=_mesh(),
        compiler_params=pltpu.CompilerParams(use_tc_tiling_on_sc=False),
        scratch_types=[
            pltpu.VMEM((CPB, 128), jnp.int32),
            pltpu.VMEM((CPB, 128), jnp.int32),
            pltpu.VMEM((8 * 128, DH), jnp.float32),
            pltpu.VMEM((320, DH), jnp.float32),
            pltpu.VMEM_SHARED((NP, DH), jnp.float32),
        ] + [pltpu.SemaphoreType.DMA] * 16,
    )
    def k(tu0_r, tu1_r, ti0_r, ti1_r, us_r, ud_r, is_r, id_r, z_r,
          accu0_r, accu1_r, acci0_r, acci1_r, six, dix, msg, stage, acc_s,
          *sems):
        c = lax.axis_index("c")
        s = lax.axis_index("s")
        off = s * RPT
        base = s * ROWS_PER_TILE
        gsem = sems[:8]
        ssem = sems[8:]

        def zero_acc():
            pltpu.sync_copy(z_r, stage)
            pltpu.sync_copy(stage, acc_s.at[pl.ds(off, 320)])
            pltpu.sync_copy(stage, acc_s.at[pl.ds(off + 320, 320)])

        def scatter_pass(tbl, src2d, dst2d):
            def mslice(j):
                return msg.at[pl.ds((j % 8) * 128, 128)]

            def blk(b, carry):
                r0 = base + b * CPB
                pltpu.sync_copy(src2d.at[pl.ds(r0, CPB)], six)
                pltpu.sync_copy(dst2d.at[pl.ds(r0, CPB)], dix)
                gd = [None] * CPB
                sd = [None] * CPB
                for j in range(4):
                    gd[j] = pltpu.async_copy(tbl.at[six.at[j]], mslice(j), gsem[j % 8])
                for j in range(CPB):
                    gd[j].wait()
                    sd[j] = pltpu.async_copy(
                        mslice(j), acc_s.at[dix.at[j]], ssem[j % 8], add=True)
                    nj = j + 4
                    if nj < CPB:
                        if nj >= 8:
                            sd[nj - 8].wait()
                        gd[nj] = pltpu.async_copy(
                            tbl.at[six.at[nj]], mslice(nj), gsem[nj % 8])
                for j in range(CPB - 8, CPB):
                    sd[j].wait()
                return carry

            lax.fori_loop(0, BLKS, blk, 0)

        def wout(out_r):
            for h in range(2):
                o = off + h * 320
                pltpu.sync_copy(acc_s.at[pl.ds(o, 320)], stage)
                pltpu.sync_copy(stage, out_r.at[pl.ds(o, 320)])

        def direction(tbl, src2d, dst2d, out_r):
            zero_acc()
            plsc.subcore_barrier()
            scatter_pass(tbl, src2d, dst2d)
            plsc.subcore_barrier()
            wout(out_r)

        @pl.when(c == 0)
        def _():
            direction(tu0_r, us_r, ud_r, acci0_r)
            direction(ti0_r, is_r, id_r, accu0_r)

        @pl.when(c == 1)
        def _():
            direction(tu1_r, us_r, ud_r, acci1_r)
            direction(ti1_r, is_r, id_r, accu1_r)

    return k(tu0, tu1, ti0, ti1, us2d, ud2d, is2d, id2d, z320)


_GRID = NP // RPT
_MATH = pl.BlockSpec((RPT, DH), lambda i: (i, 0))
_VEC = pl.BlockSpec((RPT, 1), lambda i: (i, 0))
_HALF = jax.ShapeDtypeStruct((NP, DH), jnp.float32)
_VECS = jax.ShapeDtypeStruct((NP, 1), jnp.float32)


def _tc_scales(dus, dud, dis, did, ux0, ux1, ix0, ix1):
    def body(dus_r, dud_r, dis_r, did_r, ux0_r, ux1_r, ix0_r, ix1_r,
             tu0_o, tu1_o, ti0_o, ti1_o, bu_o, bi_o, su_o, si_o,
             pu0_o, pu1_o, pi0_o, pi1_o):
        rs = lambda v: lax.rsqrt(jnp.maximum(v, 1.0))
        a_u = rs(dus_r[...])
        b_i = rs(dud_r[...])
        a_i = rs(dis_r[...])
        b_u = rs(did_r[...])
        tu0_o[...] = a_u * ux0_r[...]
        tu1_o[...] = a_u * ux1_r[...]
        ti0_o[...] = a_i * ix0_r[...]
        ti1_o[...] = a_i * ix1_r[...]
        bu_o[...] = b_u
        bi_o[...] = b_i
        su_o[...] = a_u * b_u
        si_o[...] = a_i * b_i
        pu0_o[...] = 0.25 * ux0_r[...]
        pu1_o[...] = 0.25 * ux1_r[...]
        pi0_o[...] = 0.25 * ix0_r[...]
        pi1_o[...] = 0.25 * ix1_r[...]

    return pl.pallas_call(
        body,
        grid=(_GRID,),
        in_specs=[_VEC, _VEC, _VEC, _VEC, _MATH, _MATH, _MATH, _MATH],
        out_specs=(_MATH,) * 4 + (_VEC,) * 4 + (_MATH,) * 4,
        out_shape=(_HALF,) * 4 + (_VECS,) * 4 + (_HALF,) * 4,
    )(dus, dud, dis, did, ux0, ux1, ix0, ix1)


def _tc_rescale(accu0, accu1, acci0, acci1, bu, bi, su, si,
                pu0, pu1, pi0, pi1, with_tables):
    def body(au0_r, au1_r, ai0_r, ai1_r, bu_r, bi_r, su_r, si_r,
             pu0_r, pu1_r, pi0_r, pi1_r, *outs):
        bu_v, bi_v = bu_r[...], bi_r[...]
        outs[0][...] = pu0_r[...] + 0.25 * (bu_v * au0_r[...])
        outs[1][...] = pu1_r[...] + 0.25 * (bu_v * au1_r[...])
        outs[2][...] = pi0_r[...] + 0.25 * (bi_v * ai0_r[...])
        outs[3][...] = pi1_r[...] + 0.25 * (bi_v * ai1_r[...])
        if with_tables:
            su_v, si_v = su_r[...], si_r[...]
            outs[4][...] = su_v * au0_r[...]
            outs[5][...] = su_v * au1_r[...]
            outs[6][...] = si_v * ai0_r[...]
            outs[7][...] = si_v * ai1_r[...]

    n_out = 8 if with_tables else 4
    return pl.pallas_call(
        body,
        grid=(_GRID,),
        in_specs=[_MATH] * 4 + [_VEC] * 4 + [_MATH] * 4,
        out_specs=(_MATH,) * n_out,
        out_shape=(_HALF,) * n_out,
    )(accu0, accu1, acci0, acci1, bu, bi, su, si, pu0, pu1, pi0, pi1)


def kernel(usr_x, itm_x, usr_edge_index, itm_edge_index):
    uxp = jnp.pad(usr_x, ((0, NP - N), (0, 0)))
    ixp = jnp.pad(itm_x, ((0, NP - N), (0, 0)))
    ux0, ux1 = uxp[:, :DH], uxp[:, DH:]
    ix0, ix1 = ixp[:, :DH], ixp[:, DH:]
    pad = N + (jnp.arange(EPAD - E, dtype=jnp.int32) % (NP - N))

    def prep(row):
        return jnp.concatenate([row, pad]).reshape(ROWS2D, 128)

    us2d, ud2d = prep(usr_edge_index[0]), prep(usr_edge_index[1])
    is2d, id2d = prep(itm_edge_index[0]), prep(itm_edge_index[1])
    z320 = jnp.zeros((320, DH), jnp.float32)

    dus, dud, dis, did = _sc_degrees(us2d, ud2d, is2d, id2d)
    (tu0, tu1, ti0, ti1, bu, bi, su, si,
     pu0, pu1, pi0, pi1) = _tc_scales(
        dus.reshape(NP, 1), dud.reshape(NP, 1),
        dis.reshape(NP, 1), did.reshape(NP, 1), ux0, ux1, ix0, ix1)

    for r in range(3):
        accu0, accu1, acci0, acci1 = _sc_prop(
            tu0, tu1, ti0, ti1, us2d, ud2d, is2d, id2d, z320)
        if r < 2:
            (pu0, pu1, pi0, pi1, tu0, tu1, ti0, ti1) = _tc_rescale(
                accu0, accu1, acci0, acci1, bu, bi, su, si,
                pu0, pu1, pi0, pi1, True)
        else:
            pu0, pu1, pi0, pi1 = _tc_rescale(
                accu0, accu1, acci0, acci1, bu, bi, su, si,
                pu0, pu1, pi0, pi1, False)

    new_usr = jnp.concatenate([pu0, pu1], axis=1)[:N]
    new_itm = jnp.concatenate([pi0, pi1], axis=1)[:N]
    return new_usr, new_itm


# speedup vs baseline: 14.9198x; 1.0243x over previous
import functools

import jax
import jax.numpy as jnp
from jax import lax
from jax.experimental import pallas as pl
from jax.experimental.pallas import tpu as pltpu
from jax.experimental.pallas import tpu_sc as plsc

N = 10000
NP = 10240
D = 128
DH = 64
E = 320000
NC, NS = 2, 16
CPB = 16
ROWS_PER_TILE = 160
BLKS = ROWS_PER_TILE // CPB
EPAD = ROWS_PER_TILE * 128 * NS
ROWS2D = EPAD // 128
RPT = 640


def _mesh():
    return plsc.VectorSubcoreMesh(
        core_axis_name="c", subcore_axis_name="s", num_cores=NC, num_subcores=NS
    )


def _sc_degrees(us2d, ud2d, is2d, id2d):
    @functools.partial(
        pl.kernel,
        out_type=tuple(jax.ShapeDtypeStruct((NP,), jnp.float32) for _ in range(4)),
        mesh=_mesh(),
        scratch_types=[
            pltpu.VMEM((CPB, 128), jnp.int32),
            pltpu.VMEM((CPB, 128), jnp.int32),
            pltpu.VMEM((128,), jnp.float32),
            pltpu.VMEM((RPT,), jnp.float32),
            pltpu.VMEM_SHARED((NP,), jnp.float32),
            pltpu.VMEM_SHARED((NP,), jnp.float32),
            pltpu.SemaphoreType.DMA,
            pltpu.SemaphoreType.DMA,
        ],
    )
    def k(us_r, ud_r, is_r, id_r, dus_r, dud_r, dis_r, did_r,
          six, dix, ones_v, stage_v, hist_s, hist_d, hsem, dsem):
        c = lax.axis_index("c")
        s = lax.axis_index("s")
        for kk in range(8):
            ones_v[pl.ds(kk * 16, 16)] = jnp.ones((16,), jnp.float32)
        for kk in range(RPT // 16):
            stage_v[pl.ds(kk * 16, 16)] = jnp.zeros((16,), jnp.float32)
        off = s * RPT
        pltpu.sync_copy(stage_v, hist_s.at[pl.ds(off, RPT)])
        pltpu.sync_copy(stage_v, hist_d.at[pl.ds(off, RPT)])
        plsc.subcore_barrier()

        def run(src2d, dst2d):
            base = s * ROWS_PER_TILE

            def blk(b, carry):
                r0 = base + b * CPB
                pltpu.sync_copy(src2d.at[pl.ds(r0, CPB)], six)
                pltpu.sync_copy(dst2d.at[pl.ds(r0, CPB)], dix)
                ds_ = []
                for j in range(CPB):
                    ds_.append(pltpu.async_copy(
                        ones_v, hist_s.at[six.at[j]], hsem, add=True))
                    ds_.append(pltpu.async_copy(
                        ones_v, hist_d.at[dix.at[j]], dsem, add=True))
                for d in ds_:
                    d.wait()
                return carry

            lax.fori_loop(0, BLKS, blk, 0)

        @pl.when(c == 0)
        def _():
            run(us_r, ud_r)

        @pl.when(c == 1)
        def _():
            run(is_r, id_r)

        plsc.subcore_barrier()

        def wout(hist, out_r):
            pltpu.sync_copy(hist.at[pl.ds(off, RPT)], stage_v)
            pltpu.sync_copy(stage_v, out_r.at[pl.ds(off, RPT)])

        @pl.when(c == 0)
        def _():
            wout(hist_s, dus_r)
            wout(hist_d, dud_r)

        @pl.when(c == 1)
        def _():
            wout(hist_s, dis_r)
            wout(hist_d, did_r)

    return k(us2d, ud2d, is2d, id2d)


def _sc_prop(tu0, tu1, ti0, ti1, us2d, ud2d, is2d, id2d, z320):
    half = jax.ShapeDtypeStruct((NP, DH), jnp.float32)

    @functools.partial(
        pl.kernel,
        out_type=(half, half, half, half),
        mesh=_mesh(),
        compiler_params=pltpu.CompilerParams(use_tc_tiling_on_sc=False),
        scratch_types=[
            pltpu.VMEM((CPB, 128), jnp.int32),
            pltpu.VMEM((CPB, 128), jnp.int32),
            pltpu.VMEM((8 * 128, DH), jnp.float32),
            pltpu.VMEM((320, DH), jnp.float32),
            pltpu.VMEM_SHARED((NP, DH), jnp.float32),
        ] + [pltpu.SemaphoreType.DMA] * 16,
    )
    def k(tu0_r, tu1_r, ti0_r, ti1_r, us_r, ud_r, is_r, id_r, z_r,
          accu0_r, accu1_r, acci0_r, acci1_r, six, dix, msg, stage, acc_s,
          *sems):
        c = lax.axis_index("c")
        s = lax.axis_index("s")
        off = s * RPT
        base = s * ROWS_PER_TILE
        gsem = sems[:8]
        ssem = sems[8:]

        def zero_acc():
            pltpu.sync_copy(z_r, stage)
            pltpu.sync_copy(stage, acc_s.at[pl.ds(off, 320)])
            pltpu.sync_copy(stage, acc_s.at[pl.ds(off + 320, 320)])

        def scatter_pass(tbl, src2d, dst2d):
            def mslice(j):
                return msg.at[pl.ds((j % 8) * 128, 128)]

            def blk(b, carry):
                r0 = base + b * CPB
                pltpu.sync_copy(src2d.at[pl.ds(r0, CPB)], six)
                pltpu.sync_copy(dst2d.at[pl.ds(r0, CPB)], dix)
                gd = [None] * CPB
                sd = [None] * CPB
                for j in range(4):
                    gd[j] = pltpu.async_copy(tbl.at[six.at[j]], mslice(j), gsem[j % 8])
                for j in range(CPB):
                    gd[j].wait()
                    sd[j] = pltpu.async_copy(
                        mslice(j), acc_s.at[dix.at[j]], ssem[j % 8], add=True)
                    nj = j + 4
                    if nj < CPB:
                        if nj >= 8:
                            sd[nj - 8].wait()
                        gd[nj] = pltpu.async_copy(
                            tbl.at[six.at[nj]], mslice(nj), gsem[nj % 8])
                for j in range(CPB - 8, CPB):
                    sd[j].wait()
                return carry

            lax.fori_loop(0, BLKS, blk, 0)

        def wout(out_r):
            for h in range(2):
                o = off + h * 320
                pltpu.sync_copy(acc_s.at[pl.ds(o, 320)], stage)
                pltpu.sync_copy(stage, out_r.at[pl.ds(o, 320)])

        def direction(tbl, src2d, dst2d, out_r):
            zero_acc()
            plsc.subcore_barrier()
            scatter_pass(tbl, src2d, dst2d)
            plsc.subcore_barrier()
            wout(out_r)

        @pl.when(c == 0)
        def _():
            direction(tu0_r, us_r, ud_r, acci0_r)
            direction(ti0_r, is_r, id_r, accu0_r)

        @pl.when(c == 1)
        def _():
            direction(tu1_r, us_r, ud_r, acci1_r)
            direction(ti1_r, is_r, id_r, accu1_r)

    return k(tu0, tu1, ti0, ti1, us2d, ud2d, is2d, id2d, z320)


_GRID = NP // RPT
_MATH = pl.BlockSpec((RPT, DH), lambda i: (i, 0))
_VEC = pl.BlockSpec((RPT, 1), lambda i: (i, 0))
_HALF = jax.ShapeDtypeStruct((NP, DH), jnp.float32)
_VECS = jax.ShapeDtypeStruct((NP, 1), jnp.float32)


def _tc_scales(dus, dud, dis, did, ux0, ux1, ix0, ix1):
    def body(dus_r, dud_r, dis_r, did_r, ux0_r, ux1_r, ix0_r, ix1_r,
             tu0_o, tu1_o, ti0_o, ti1_o, bu_o, bi_o, su_o, si_o,
             pu0_o, pu1_o, pi0_o, pi1_o):
        rs = lambda v: lax.rsqrt(jnp.maximum(v, 1.0))
        a_u = rs(dus_r[...])
        b_i = rs(dud_r[...])
        a_i = rs(dis_r[...])
        b_u = rs(did_r[...])
        tu0_o[...] = a_u * ux0_r[...]
        tu1_o[...] = a_u * ux1_r[...]
        ti0_o[...] = a_i * ix0_r[...]
        ti1_o[...] = a_i * ix1_r[...]
        bu_o[...] = b_u
        bi_o[...] = b_i
        su_o[...] = a_u * b_u
        si_o[...] = a_i * b_i
        pu0_o[...] = 0.25 * ux0_r[...]
        pu1_o[...] = 0.25 * ux1_r[...]
        pi0_o[...] = 0.25 * ix0_r[...]
        pi1_o[...] = 0.25 * ix1_r[...]

    return pl.pallas_call(
        body,
        grid=(_GRID,),
        in_specs=[_VEC, _VEC, _VEC, _VEC, _MATH, _MATH, _MATH, _MATH],
        out_specs=(_MATH,) * 4 + (_VEC,) * 4 + (_MATH,) * 4,
        out_shape=(_HALF,) * 4 + (_VECS,) * 4 + (_HALF,) * 4,
    )(dus, dud, dis, did, ux0, ux1, ix0, ix1)


def _tc_rescale(accu0, accu1, acci0, acci1, bu, bi, su, si,
                pu0, pu1, pi0, pi1, with_tables):
    def body(au0_r, au1_r, ai0_r, ai1_r, bu_r, bi_r, su_r, si_r,
             pu0_r, pu1_r, pi0_r, pi1_r, *outs):
        bu_v, bi_v = bu_r[...], bi_r[...]
        outs[0][...] = pu0_r[...] + 0.25 * (bu_v * au0_r[...])
        outs[1][...] = pu1_r[...] + 0.25 * (bu_v * au1_r[...])
        outs[2][...] = pi0_r[...] + 0.25 * (bi_v * ai0_r[...])
        outs[3][...] = pi1_r[...] + 0.25 * (bi_v * ai1_r[...])
        if with_tables:
            su_v, si_v = su_r[...], si_r[...]
            outs[4][...] = su_v * au0_r[...]
            outs[5][...] = su_v * au1_r[...]
            outs[6][...] = si_v * ai0_r[...]
            outs[7][...] = si_v * ai1_r[...]

    n_out = 8 if with_tables else 4
    return pl.pallas_call(
        body,
        grid=(_GRID,),
        in_specs=[_MATH] * 4 + [_VEC] * 4 + [_MATH] * 4,
        out_specs=(_MATH,) * n_out,
        out_shape=(_HALF,) * n_out,
    )(accu0, accu1, acci0, acci1, bu, bi, su, si, pu0, pu1, pi0, pi1)


def kernel(usr_x, itm_x, usr_edge_index, itm_edge_index):
    uxp = jnp.pad(usr_x, ((0, NP - N), (0, 0)))
    ixp = jnp.pad(itm_x, ((0, NP - N), (0, 0)))
    ux0, ux1 = uxp[:, :DH], uxp[:, DH:]
    ix0, ix1 = ixp[:, :DH], ixp[:, DH:]
    pad = N + (jnp.arange(EPAD - E, dtype=jnp.int32) % (NP - N))

    def prep(row):
        return jnp.concatenate([row, pad]).reshape(ROWS2D, 128)

    us2d, ud2d = prep(usr_edge_index[0]), prep(usr_edge_index[1])
    is2d, id2d = prep(itm_edge_index[0]), prep(itm_edge_index[1])
    z320 = jnp.zeros((320, DH), jnp.float32)

    dus, dud, dis, did = _sc_degrees(us2d, ud2d, is2d, id2d)
    (tu0, tu1, ti0, ti1, bu, bi, su, si,
     pu0, pu1, pi0, pi1) = _tc_scales(
        dus.reshape(NP, 1), dud.reshape(NP, 1),
        dis.reshape(NP, 1), did.reshape(NP, 1), ux0, ux1, ix0, ix1)

    for r in range(3):
        accu0, accu1, acci0, acci1 = _sc_prop(
            tu0, tu1, ti0, ti1, us2d, ud2d, is2d, id2d, z320)
        if r < 2:
            (pu0, pu1, pi0, pi1, tu0, tu1, ti0, ti1) = _tc_rescale(
                accu0, accu1, acci0, acci1, bu, bi, su, si,
                pu0, pu1, pi0, pi1, True)
        else:
            pu0, pu1, pi0, pi1 = _tc_rescale(
                accu0, accu1, acci0, acci1, bu, bi, su, si,
                pu0, pu1, pi0, pi1, False)

    new_usr = jnp.concatenate([pu0, pu1], axis=1)[:N]
    new_itm = jnp.concatenate([pi0, pi1], axis=1)[:N]
    return new_usr, new_itm


# speedup vs baseline: 15.2715x; 1.0236x over previous
import functools

import jax
import jax.numpy as jnp
from jax import lax
from jax.experimental import pallas as pl
from jax.experimental.pallas import tpu as pltpu
from jax.experimental.pallas import tpu_sc as plsc

N = 10000
NP = 10112
D = 128
DH = 64
E = 320000
NC, NS = 2, 16
CPB = 16
ROWS_PER_TILE = 160
BLKS = ROWS_PER_TILE // CPB
EPAD = ROWS_PER_TILE * 128 * NS
ROWS2D = EPAD // 128
RPT = 632
HBA, HBB = 312, 320


def _mesh():
    return plsc.VectorSubcoreMesh(
        core_axis_name="c", subcore_axis_name="s", num_cores=NC, num_subcores=NS
    )


def _sc_degrees(us2d, ud2d, is2d, id2d):
    @functools.partial(
        pl.kernel,
        out_type=tuple(jax.ShapeDtypeStruct((NP,), jnp.float32) for _ in range(4)),
        mesh=_mesh(),
        scratch_types=[
            pltpu.VMEM((CPB, 128), jnp.int32),
            pltpu.VMEM((CPB, 128), jnp.int32),
            pltpu.VMEM((128,), jnp.float32),
            pltpu.VMEM((RPT,), jnp.float32),
            pltpu.VMEM_SHARED((NP,), jnp.float32),
            pltpu.VMEM_SHARED((NP,), jnp.float32),
            pltpu.SemaphoreType.DMA,
            pltpu.SemaphoreType.DMA,
        ],
    )
    def k(us_r, ud_r, is_r, id_r, dus_r, dud_r, dis_r, did_r,
          six, dix, ones_v, stage_v, hist_s, hist_d, hsem, dsem):
        c = lax.axis_index("c")
        s = lax.axis_index("s")
        for kk in range(8):
            ones_v[pl.ds(kk * 16, 16)] = jnp.ones((16,), jnp.float32)
        for kk in range(RPT // 16):
            stage_v[pl.ds(kk * 16, 16)] = jnp.zeros((16,), jnp.float32)
        off = s * RPT
        pltpu.sync_copy(stage_v, hist_s.at[pl.ds(off, RPT)])
        pltpu.sync_copy(stage_v, hist_d.at[pl.ds(off, RPT)])
        plsc.subcore_barrier()

        def run(src2d, dst2d):
            base = s * ROWS_PER_TILE

            def blk(b, carry):
                r0 = base + b * CPB
                pltpu.sync_copy(src2d.at[pl.ds(r0, CPB)], six)
                pltpu.sync_copy(dst2d.at[pl.ds(r0, CPB)], dix)
                ds_ = []
                for j in range(CPB):
                    ds_.append(pltpu.async_copy(
                        ones_v, hist_s.at[six.at[j]], hsem, add=True))
                    ds_.append(pltpu.async_copy(
                        ones_v, hist_d.at[dix.at[j]], dsem, add=True))
                for d in ds_:
                    d.wait()
                return carry

            lax.fori_loop(0, BLKS, blk, 0)

        @pl.when(c == 0)
        def _():
            run(us_r, ud_r)

        @pl.when(c == 1)
        def _():
            run(is_r, id_r)

        plsc.subcore_barrier()

        def wout(hist, out_r):
            pltpu.sync_copy(hist.at[pl.ds(off, RPT)], stage_v)
            pltpu.sync_copy(stage_v, out_r.at[pl.ds(off, RPT)])

        @pl.when(c == 0)
        def _():
            wout(hist_s, dus_r)
            wout(hist_d, dud_r)

        @pl.when(c == 1)
        def _():
            wout(hist_s, dis_r)
            wout(hist_d, did_r)

    return k(us2d, ud2d, is2d, id2d)


def _sc_prop(tu0, tu1, ti0, ti1, us2d, ud2d, is2d, id2d, z320):
    half = jax.ShapeDtypeStruct((NP, DH), jnp.float32)

    @functools.partial(
        pl.kernel,
        out_type=(half, half, half, half),
        mesh=_mesh(),
        compiler_params=pltpu.CompilerParams(use_tc_tiling_on_sc=False),
        scratch_types=[
            pltpu.VMEM((CPB, 128), jnp.int32),
            pltpu.VMEM((CPB, 128), jnp.int32),
            pltpu.VMEM((8 * 128, DH), jnp.float32),
            pltpu.VMEM((HBB, DH), jnp.float32),
            pltpu.VMEM_SHARED((NP, DH), jnp.float32),
        ] + [pltpu.SemaphoreType.DMA] * 16,
    )
    def k(tu0_r, tu1_r, ti0_r, ti1_r, us_r, ud_r, is_r, id_r, z_r,
          accu0_r, accu1_r, acci0_r, acci1_r, six, dix, msg, stage, acc_s,
          *sems):
        c = lax.axis_index("c")
        s = lax.axis_index("s")
        off = s * RPT
        base = s * ROWS_PER_TILE
        gsem = sems[:8]
        ssem = sems[8:]

        def zero_acc():
            pltpu.sync_copy(z_r, stage)
            pltpu.sync_copy(stage.at[pl.ds(0, HBA)], acc_s.at[pl.ds(off, HBA)])
            pltpu.sync_copy(stage, acc_s.at[pl.ds(off + HBA, HBB)])

        def scatter_pass(tbl, src2d, dst2d):
            def mslice(j):
                return msg.at[pl.ds((j % 8) * 128, 128)]

            def blk(b, carry):
                r0 = base + b * CPB
                pltpu.sync_copy(src2d.at[pl.ds(r0, CPB)], six)
                pltpu.sync_copy(dst2d.at[pl.ds(r0, CPB)], dix)
                gd = [None] * CPB
                sd = [None] * CPB
                for j in range(6):
                    gd[j] = pltpu.async_copy(tbl.at[six.at[j]], mslice(j), gsem[j % 8])
                for j in range(CPB):
                    gd[j].wait()
                    sd[j] = pltpu.async_copy(
                        mslice(j), acc_s.at[dix.at[j]], ssem[j % 8], add=True)
                    nj = j + 6
                    if nj < CPB:
                        if nj >= 8:
                            sd[nj - 8].wait()
                        gd[nj] = pltpu.async_copy(
                            tbl.at[six.at[nj]], mslice(nj), gsem[nj % 8])
                for j in range(CPB - 8, CPB):
                    sd[j].wait()
                return carry

            lax.fori_loop(0, BLKS, blk, 0)

        def wout(out_r):
            pltpu.sync_copy(acc_s.at[pl.ds(off, HBA)], stage.at[pl.ds(0, HBA)])
            pltpu.sync_copy(stage.at[pl.ds(0, HBA)], out_r.at[pl.ds(off, HBA)])
            pltpu.sync_copy(acc_s.at[pl.ds(off + HBA, HBB)], stage)
            pltpu.sync_copy(stage, out_r.at[pl.ds(off + HBA, HBB)])

        def direction(tbl, src2d, dst2d, out_r):
            zero_acc()
            plsc.subcore_barrier()
            scatter_pass(tbl, src2d, dst2d)
            plsc.subcore_barrier()
            wout(out_r)

        @pl.when(c == 0)
        def _():
            direction(tu0_r, us_r, ud_r, acci0_r)
            direction(ti0_r, is_r, id_r, accu0_r)

        @pl.when(c == 1)
        def _():
            direction(tu1_r, us_r, ud_r, acci1_r)
            direction(ti1_r, is_r, id_r, accu1_r)

    return k(tu0, tu1, ti0, ti1, us2d, ud2d, is2d, id2d, z320)


_GRID = NP // RPT
_MATH = pl.BlockSpec((RPT, DH), lambda i: (i, 0))
_VEC = pl.BlockSpec((RPT, 1), lambda i: (i, 0))
_HALF = jax.ShapeDtypeStruct((NP, DH), jnp.float32)
_VECS = jax.ShapeDtypeStruct((NP, 1), jnp.float32)


def _tc_scales(dus, dud, dis, did, ux0, ux1, ix0, ix1):
    def body(dus_r, dud_r, dis_r, did_r, ux0_r, ux1_r, ix0_r, ix1_r,
             tu0_o, tu1_o, ti0_o, ti1_o, bu_o, bi_o, su_o, si_o,
             pu0_o, pu1_o, pi0_o, pi1_o):
        rs = lambda v: lax.rsqrt(jnp.maximum(v, 1.0))
        a_u = rs(dus_r[...])
        b_i = rs(dud_r[...])
        a_i = rs(dis_r[...])
        b_u = rs(did_r[...])
        tu0_o[...] = a_u * ux0_r[...]
        tu1_o[...] = a_u * ux1_r[...]
        ti0_o[...] = a_i * ix0_r[...]
        ti1_o[...] = a_i * ix1_r[...]
        bu_o[...] = b_u
        bi_o[...] = b_i
        su_o[...] = a_u * b_u
        si_o[...] = a_i * b_i
        pu0_o[...] = 0.25 * ux0_r[...]
        pu1_o[...] = 0.25 * ux1_r[...]
        pi0_o[...] = 0.25 * ix0_r[...]
        pi1_o[...] = 0.25 * ix1_r[...]

    return pl.pallas_call(
        body,
        grid=(_GRID,),
        in_specs=[_VEC, _VEC, _VEC, _VEC, _MATH, _MATH, _MATH, _MATH],
        out_specs=(_MATH,) * 4 + (_VEC,) * 4 + (_MATH,) * 4,
        out_shape=(_HALF,) * 4 + (_VECS,) * 4 + (_HALF,) * 4,
    )(dus, dud, dis, did, ux0, ux1, ix0, ix1)


def _tc_rescale(accu0, accu1, acci0, acci1, bu, bi, su, si,
                pu0, pu1, pi0, pi1, with_tables):
    def body(au0_r, au1_r, ai0_r, ai1_r, bu_r, bi_r, su_r, si_r,
             pu0_r, pu1_r, pi0_r, pi1_r, *outs):
        bu_v, bi_v = bu_r[...], bi_r[...]
        outs[0][...] = pu0_r[...] + 0.25 * (bu_v * au0_r[...])
        outs[1][...] = pu1_r[...] + 0.25 * (bu_v * au1_r[...])
        outs[2][...] = pi0_r[...] + 0.25 * (bi_v * ai0_r[...])
        outs[3][...] = pi1_r[...] + 0.25 * (bi_v * ai1_r[...])
        if with_tables:
            su_v, si_v = su_r[...], si_r[...]
            outs[4][...] = su_v * au0_r[...]
            outs[5][...] = su_v * au1_r[...]
            outs[6][...] = si_v * ai0_r[...]
            outs[7][...] = si_v * ai1_r[...]

    n_out = 8 if with_tables else 4
    return pl.pallas_call(
        body,
        grid=(_GRID,),
        in_specs=[_MATH] * 4 + [_VEC] * 4 + [_MATH] * 4,
        out_specs=(_MATH,) * n_out,
        out_shape=(_HALF,) * n_out,
    )(accu0, accu1, acci0, acci1, bu, bi, su, si, pu0, pu1, pi0, pi1)


def kernel(usr_x, itm_x, usr_edge_index, itm_edge_index):
    uxp = jnp.pad(usr_x, ((0, NP - N), (0, 0)))
    ixp = jnp.pad(itm_x, ((0, NP - N), (0, 0)))
    ux0, ux1 = uxp[:, :DH], uxp[:, DH:]
    ix0, ix1 = ixp[:, :DH], ixp[:, DH:]
    pad = N + (jnp.arange(EPAD - E, dtype=jnp.int32) % (NP - N))

    def prep(row):
        return jnp.concatenate([row, pad]).reshape(ROWS2D, 128)

    us2d, ud2d = prep(usr_edge_index[0]), prep(usr_edge_index[1])
    is2d, id2d = prep(itm_edge_index[0]), prep(itm_edge_index[1])
    z320 = jnp.zeros((HBB, DH), jnp.float32)

    dus, dud, dis, did = _sc_degrees(us2d, ud2d, is2d, id2d)
    (tu0, tu1, ti0, ti1, bu, bi, su, si,
     pu0, pu1, pi0, pi1) = _tc_scales(
        dus.reshape(NP, 1), dud.reshape(NP, 1),
        dis.reshape(NP, 1), did.reshape(NP, 1), ux0, ux1, ix0, ix1)

    for r in range(3):
        accu0, accu1, acci0, acci1 = _sc_prop(
            tu0, tu1, ti0, ti1, us2d, ud2d, is2d, id2d, z320)
        if r < 2:
            (pu0, pu1, pi0, pi1, tu0, tu1, ti0, ti1) = _tc_rescale(
                accu0, accu1, acci0, acci1, bu, bi, su, si,
                pu0, pu1, pi0, pi1, True)
        else:
            pu0, pu1, pi0, pi1 = _tc_rescale(
                accu0, accu1, acci0, acci1, bu, bi, su, si,
                pu0, pu1, pi0, pi1, False)

    new_usr = jnp.concatenate([pu0, pu1], axis=1)[:N]
    new_itm = jnp.concatenate([pi0, pi1], axis=1)[:N]
    return new_usr, new_itm


# speedup vs baseline: 15.2834x; 1.0008x over previous
import functools

import jax
import jax.numpy as jnp
from jax import lax
from jax.experimental import pallas as pl
from jax.experimental.pallas import tpu as pltpu
from jax.experimental.pallas import tpu_sc as plsc

N = 10000
NP = 10112
D = 128
DH = 64
E = 320000
NC, NS = 2, 16
CPB = 16
ROWS_PER_TILE = 160
BLKS = ROWS_PER_TILE // CPB
EPAD = ROWS_PER_TILE * 128 * NS
ROWS2D = EPAD // 128
RPT = 632
HBA, HBB = 312, 320


def _mesh():
    return plsc.VectorSubcoreMesh(
        core_axis_name="c", subcore_axis_name="s", num_cores=NC, num_subcores=NS
    )


def _sc_degrees(us2d, ud2d, is2d, id2d):
    @functools.partial(
        pl.kernel,
        out_type=tuple(jax.ShapeDtypeStruct((NP,), jnp.float32) for _ in range(4)),
        mesh=_mesh(),
        scratch_types=[
            pltpu.VMEM((CPB, 128), jnp.int32),
            pltpu.VMEM((CPB, 128), jnp.int32),
            pltpu.VMEM((128,), jnp.float32),
            pltpu.VMEM((640,), jnp.float32),
            pltpu.VMEM_SHARED((NP,), jnp.float32),
            pltpu.VMEM_SHARED((NP,), jnp.float32),
            pltpu.SemaphoreType.DMA,
            pltpu.SemaphoreType.DMA,
        ],
    )
    def k(us_r, ud_r, is_r, id_r, dus_r, dud_r, dis_r, did_r,
          six, dix, ones_v, stage_v, hist_s, hist_d, hsem, dsem):
        c = lax.axis_index("c")
        s = lax.axis_index("s")
        for kk in range(8):
            ones_v[pl.ds(kk * 16, 16)] = jnp.ones((16,), jnp.float32)
        for kk in range(640 // 16):
            stage_v[pl.ds(kk * 16, 16)] = jnp.zeros((16,), jnp.float32)
        off = s * RPT
        pltpu.sync_copy(stage_v.at[pl.ds(0, RPT)], hist_s.at[pl.ds(off, RPT)])
        pltpu.sync_copy(stage_v.at[pl.ds(0, RPT)], hist_d.at[pl.ds(off, RPT)])
        plsc.subcore_barrier()

        def run(src2d, dst2d):
            base = s * ROWS_PER_TILE

            def blk(b, carry):
                r0 = base + b * CPB
                pltpu.sync_copy(src2d.at[pl.ds(r0, CPB)], six)
                pltpu.sync_copy(dst2d.at[pl.ds(r0, CPB)], dix)
                ds_ = []
                for j in range(CPB):
                    ds_.append(pltpu.async_copy(
                        ones_v, hist_s.at[six.at[j]], hsem, add=True))
                    ds_.append(pltpu.async_copy(
                        ones_v, hist_d.at[dix.at[j]], dsem, add=True))
                for d in ds_:
                    d.wait()
                return carry

            lax.fori_loop(0, BLKS, blk, 0)

        @pl.when(c == 0)
        def _():
            run(us_r, ud_r)

        @pl.when(c == 1)
        def _():
            run(is_r, id_r)

        plsc.subcore_barrier()

        def wout(hist, out_r):
            pltpu.sync_copy(hist.at[pl.ds(off, RPT)], stage_v.at[pl.ds(0, RPT)])
            pltpu.sync_copy(stage_v.at[pl.ds(0, RPT)], out_r.at[pl.ds(off, RPT)])

        @pl.when(c == 0)
        def _():
            wout(hist_s, dus_r)
            wout(hist_d, dud_r)

        @pl.when(c == 1)
        def _():
            wout(hist_s, dis_r)
            wout(hist_d, did_r)

    return k(us2d, ud2d, is2d, id2d)


def _sc_prop(tu0, tu1, ti0, ti1, us2d, ud2d, is2d, id2d, z320):
    half = jax.ShapeDtypeStruct((NP, DH), jnp.float32)

    @functools.partial(
        pl.kernel,
        out_type=(half, half, half, half),
        mesh=_mesh(),
        compiler_params=pltpu.CompilerParams(use_tc_tiling_on_sc=False),
        scratch_types=[
            pltpu.VMEM((CPB, 128), jnp.int32),
            pltpu.VMEM((CPB, 128), jnp.int32),
            pltpu.VMEM((8 * 128, DH), jnp.float32),
            pltpu.VMEM((HBB, DH), jnp.float32),
            pltpu.VMEM_SHARED((NP, DH), jnp.float32),
        ] + [pltpu.SemaphoreType.DMA] * 16,
    )
    def k(tu0_r, tu1_r, ti0_r, ti1_r, us_r, ud_r, is_r, id_r, z_r,
          accu0_r, accu1_r, acci0_r, acci1_r, six, dix, msg, stage, acc_s,
          *sems):
        c = lax.axis_index("c")
        s = lax.axis_index("s")
        off = s * RPT
        base = s * ROWS_PER_TILE
        gsem = sems[:8]
        ssem = sems[8:]

        def zero_acc():
            pltpu.sync_copy(z_r, stage)
            pltpu.sync_copy(stage.at[pl.ds(0, HBA)], acc_s.at[pl.ds(off, HBA)])
            pltpu.sync_copy(stage, acc_s.at[pl.ds(off + HBA, HBB)])

        def scatter_pass(tbl, src2d, dst2d):
            def mslice(j):
                return msg.at[pl.ds((j % 8) * 128, 128)]

            def blk(b, carry):
                r0 = base + b * CPB
                pltpu.sync_copy(src2d.at[pl.ds(r0, CPB)], six)
                pltpu.sync_copy(dst2d.at[pl.ds(r0, CPB)], dix)
                gd = [None] * CPB
                sd = [None] * CPB
                for j in range(6):
                    gd[j] = pltpu.async_copy(tbl.at[six.at[j]], mslice(j), gsem[j % 8])
                for j in range(CPB):
                    gd[j].wait()
                    sd[j] = pltpu.async_copy(
                        mslice(j), acc_s.at[dix.at[j]], ssem[j % 8], add=True)
                    nj = j + 6
                    if nj < CPB:
                        if nj >= 8:
                            sd[nj - 8].wait()
                        gd[nj] = pltpu.async_copy(
                            tbl.at[six.at[nj]], mslice(nj), gsem[nj % 8])
                for j in range(CPB - 8, CPB):
                    sd[j].wait()
                return carry

            lax.fori_loop(0, BLKS, blk, 0)

        def wout(out_r):
            pltpu.sync_copy(acc_s.at[pl.ds(off, HBA)], stage.at[pl.ds(0, HBA)])
            pltpu.sync_copy(stage.at[pl.ds(0, HBA)], out_r.at[pl.ds(off, HBA)])
            pltpu.sync_copy(acc_s.at[pl.ds(off + HBA, HBB)], stage)
            pltpu.sync_copy(stage, out_r.at[pl.ds(off + HBA, HBB)])

        def direction(tbl, src2d, dst2d, out_r):
            zero_acc()
            plsc.subcore_barrier()
            scatter_pass(tbl, src2d, dst2d)
            plsc.subcore_barrier()
            wout(out_r)

        @pl.when(c == 0)
        def _():
            direction(tu0_r, us_r, ud_r, acci0_r)
            direction(ti0_r, is_r, id_r, accu0_r)

        @pl.when(c == 1)
        def _():
            direction(tu1_r, us_r, ud_r, acci1_r)
            direction(ti1_r, is_r, id_r, accu1_r)

    return k(tu0, tu1, ti0, ti1, us2d, ud2d, is2d, id2d, z320)


_GRID = NP // RPT
_MATH = pl.BlockSpec((RPT, DH), lambda i: (i, 0))
_VEC = pl.BlockSpec((RPT, 1), lambda i: (i, 0))
_HALF = jax.ShapeDtypeStruct((NP, DH), jnp.float32)
_VECS = jax.ShapeDtypeStruct((NP, 1), jnp.float32)


def _tc_scales(dus, dud, dis, did, ux0, ux1, ix0, ix1):
    def body(dus_r, dud_r, dis_r, did_r, ux0_r, ux1_r, ix0_r, ix1_r,
             tu0_o, tu1_o, ti0_o, ti1_o, bu_o, bi_o, su_o, si_o,
             pu0_o, pu1_o, pi0_o, pi1_o):
        rs = lambda v: lax.rsqrt(jnp.maximum(v, 1.0))
        a_u = rs(dus_r[...])
        b_i = rs(dud_r[...])
        a_i = rs(dis_r[...])
        b_u = rs(did_r[...])
        tu0_o[...] = a_u * ux0_r[...]
        tu1_o[...] = a_u * ux1_r[...]
        ti0_o[...] = a_i * ix0_r[...]
        ti1_o[...] = a_i * ix1_r[...]
        bu_o[...] = b_u
        bi_o[...] = b_i
        su_o[...] = a_u * b_u
        si_o[...] = a_i * b_i
        pu0_o[...] = 0.25 * ux0_r[...]
        pu1_o[...] = 0.25 * ux1_r[...]
        pi0_o[...] = 0.25 * ix0_r[...]
        pi1_o[...] = 0.25 * ix1_r[...]

    return pl.pallas_call(
        body,
        grid=(_GRID,),
        in_specs=[_VEC, _VEC, _VEC, _VEC, _MATH, _MATH, _MATH, _MATH],
        out_specs=(_MATH,) * 4 + (_VEC,) * 4 + (_MATH,) * 4,
        out_shape=(_HALF,) * 4 + (_VECS,) * 4 + (_HALF,) * 4,
    )(dus, dud, dis, did, ux0, ux1, ix0, ix1)


def _tc_rescale(accu0, accu1, acci0, acci1, bu, bi, su, si,
                pu0, pu1, pi0, pi1, with_tables):
    def body(au0_r, au1_r, ai0_r, ai1_r, bu_r, bi_r, su_r, si_r,
             pu0_r, pu1_r, pi0_r, pi1_r, *outs):
        bu_v, bi_v = bu_r[...], bi_r[...]
        outs[0][...] = pu0_r[...] + 0.25 * (bu_v * au0_r[...])
        outs[1][...] = pu1_r[...] + 0.25 * (bu_v * au1_r[...])
        outs[2][...] = pi0_r[...] + 0.25 * (bi_v * ai0_r[...])
        outs[3][...] = pi1_r[...] + 0.25 * (bi_v * ai1_r[...])
        if with_tables:
            su_v, si_v = su_r[...], si_r[...]
            outs[4][...] = su_v * au0_r[...]
            outs[5][...] = su_v * au1_r[...]
            outs[6][...] = si_v * ai0_r[...]
            outs[7][...] = si_v * ai1_r[...]

    n_out = 8 if with_tables else 4
    return pl.pallas_call(
        body,
        grid=(_GRID,),
        in_specs=[_MATH] * 4 + [_VEC] * 4 + [_MATH] * 4,
        out_specs=(_MATH,) * n_out,
        out_shape=(_HALF,) * n_out,
    )(accu0, accu1, acci0, acci1, bu, bi, su, si, pu0, pu1, pi0, pi1)


def kernel(usr_x, itm_x, usr_edge_index, itm_edge_index):
    uxp = jnp.pad(usr_x, ((0, NP - N), (0, 0)))
    ixp = jnp.pad(itm_x, ((0, NP - N), (0, 0)))
    ux0, ux1 = uxp[:, :DH], uxp[:, DH:]
    ix0, ix1 = ixp[:, :DH], ixp[:, DH:]
    pad = N + (jnp.arange(EPAD - E, dtype=jnp.int32) % (NP - N))

    def prep(row):
        return jnp.concatenate([row, pad]).reshape(ROWS2D, 128)

    us2d, ud2d = prep(usr_edge_index[0]), prep(usr_edge_index[1])
    is2d, id2d = prep(itm_edge_index[0]), prep(itm_edge_index[1])
    z320 = jnp.zeros((HBB, DH), jnp.float32)

    dus, dud, dis, did = _sc_degrees(us2d, ud2d, is2d, id2d)
    (tu0, tu1, ti0, ti1, bu, bi, su, si,
     pu0, pu1, pi0, pi1) = _tc_scales(
        dus.reshape(NP, 1), dud.reshape(NP, 1),
        dis.reshape(NP, 1), did.reshape(NP, 1), ux0, ux1, ix0, ix1)

    for r in range(3):
        accu0, accu1, acci0, acci1 = _sc_prop(
            tu0, tu1, ti0, ti1, us2d, ud2d, is2d, id2d, z320)
        if r < 2:
            (pu0, pu1, pi0, pi1, tu0, tu1, ti0, ti1) = _tc_rescale(
                accu0, accu1, acci0, acci1, bu, bi, su, si,
                pu0, pu1, pi0, pi1, True)
        else:
            pu0, pu1, pi0, pi1 = _tc_rescale(
                accu0, accu1, acci0, acci1, bu, bi, su, si,
                pu0, pu1, pi0, pi1, False)

    new_usr = jnp.concatenate([pu0, pu1], axis=1)[:N]
    new_itm = jnp.concatenate([pi0, pi1], axis=1)[:N]
    return new_usr, new_itm
